# baseline (speedup 1.0000x reference)
import functools

import jax
import jax.numpy as jnp
from jax import lax
from jax.experimental import pallas as pl
from jax.experimental.pallas import tpu as pltpu
from jax.experimental.pallas import tpu_sc as plsc

N = 10000
E = 160000
R = 8
D = 256
OUT = 64

NC = 2
NS = 16
L = 16

E2 = 163840
ET2 = E2 // (NC * NS)
ET = E // NS
CB = 80
NCH = ET // CB
NCH2 = ET2 // CB
RB = 624
CPT = (N * R) // NS


def _splat(v, j):
  idx = jnp.full((L, 1), j, jnp.int32)
  dn = lax.GatherDimensionNumbers(
      offset_dims=(), collapsed_slice_dims=(0,), start_index_map=(0,))
  return lax.gather(v, idx, dn, (1,),
                    mode=lax.GatherScatterMode.PROMISE_IN_BOUNDS)


def _zero_2d(ref, nrows, ncols):
  z = jnp.zeros((L,), jnp.float32)

  def row(i, _):
    def col(k, _):
      ref[i, pl.ds(k * L, L)] = z
      return 0
    return lax.fori_loop(0, ncols // L, col, 0)
  lax.fori_loop(0, nrows, row, 0)


def _zero_1d(ref, n):
  z = jnp.zeros((L,), jnp.float32)

  def body(i, _):
    ref[pl.ds(i * L, L)] = z
    return 0
  lax.fori_loop(0, n // L, body, 0)


def _scale_rows(rows, nrmref, base, fsc):
  def group(g, _):
    n16 = nrmref[pl.ds(base + g * L, L)]
    for j in range(L):
      e = g * L + j
      ns = _splat(n16, j)
      for f in range(fsc // L):
        rows[e, pl.ds(f * L, L)] = rows[e, pl.ds(f * L, L)] * ns
    return 0
  lax.fori_loop(0, CB // L, group, 0)


def _row_range(s):
  base = s * RB
  n80 = jnp.where(s == NS - 1, 8, 7)
  tail = jnp.where(s == NS - 1, 0, 64)
  return base, n80, tail


def _zero_phase(agg_sh, rows, fh, s, cnt_sh=None, zc=None):
  _zero_2d(rows, CB, fh)
  base, n80, tail = _row_range(s)

  def body(k, _):
    pltpu.sync_copy(rows, agg_sh.at[pl.ds(base + k * CB, CB)])
    return 0
  lax.fori_loop(0, n80, body, 0)

  @pl.when(tail > 0)
  def _():
    pltpu.sync_copy(rows.at[pl.ds(0, 64)],
                    agg_sh.at[pl.ds(base + 7 * CB, 64)])
  if cnt_sh is not None:
    _zero_1d(zc, 5008)
    pltpu.sync_copy(zc.at[pl.ds(0, CPT)], cnt_sh.at[pl.ds(s * CPT, CPT)])


def _writeout(agg_sh, agg_out, rows, c, s):
  base, n80, tail = _row_range(s)

  def body(k, _):
    o = base + k * CB
    pltpu.sync_copy(agg_sh.at[pl.ds(o, CB)], rows)
    pltpu.sync_copy(rows, agg_out.at[c, pl.ds(o, CB)])
    return 0
  lax.fori_loop(0, n80, body, 0)

  @pl.when(tail > 0)
  def _():
    o = base + 7 * CB
    pltpu.sync_copy(agg_sh.at[pl.ds(o, 64)], rows.at[pl.ds(0, 64)])
    pltpu.sync_copy(rows.at[pl.ds(0, 64)], agg_out.at[c, pl.ds(o, 64)])


def _agg1_body(table, src_h, dst_h, rel_h, agg_out, norm_out,
               agg_sh, cnt_sh,
               rows0, rows1, gidx0, gidx1, didx0, didx1, kidx0, kidx1,
               ebs0, ebd0, ebr0, ebs1, ebd1, ebr1,
               ones, normall, cv0, cv1, zc,
               sem_e0, sem_e1, sem_r0, sem_r1, sem_s0, sem_s1,
               sem_c0, sem_c1):
  c = lax.axis_index("c")
  s = lax.axis_index("s")
  rows = (rows0, rows1)
  gidx = (gidx0, gidx1)
  didx = (didx0, didx1)
  kidx = (kidx0, kidx1)
  ebs = (ebs0, ebs1)
  ebd = (ebd0, ebd1)
  ebr = (ebr0, ebr1)
  cv = (cv0, cv1)
  sem_e = (sem_e0, sem_e1)
  sem_r = (sem_r0, sem_r1)
  sem_s = (sem_s0, sem_s1)
  sem_c = (sem_c0, sem_c1)

  def eoff(k):
    return s * ET + k * CB

  def issue_dr(k, b):
    o = eoff(k)
    pltpu.async_copy(dst_h.at[pl.ds(o, CB)], ebd[b], sem_e[b])
    pltpu.async_copy(rel_h.at[pl.ds(o, CB)], ebr[b], sem_e[b])

  def wait_dr(k, b):
    o = eoff(k)
    pltpu.make_async_copy(dst_h.at[pl.ds(o, CB)], ebd[b], sem_e[b]).wait()
    pltpu.make_async_copy(rel_h.at[pl.ds(o, CB)], ebr[b], sem_e[b]).wait()

  def issue_edges(k, b):
    o = eoff(k)
    pltpu.async_copy(src_h.at[pl.ds(o, CB)], ebs[b], sem_e[b])
    pltpu.async_copy(dst_h.at[pl.ds(o, CB)], ebd[b], sem_e[b])
    pltpu.async_copy(rel_h.at[pl.ds(o, CB)], ebr[b], sem_e[b])

  def wait_edges(k, b):
    o = eoff(k)
    pltpu.make_async_copy(src_h.at[pl.ds(o, CB)], ebs[b], sem_e[b]).wait()
    pltpu.make_async_copy(dst_h.at[pl.ds(o, CB)], ebd[b], sem_e[b]).wait()
    pltpu.make_async_copy(rel_h.at[pl.ds(o, CB)], ebr[b], sem_e[b]).wait()

  _zero_phase(agg_sh, rows0, 128, s, cnt_sh, zc)
  _zero_2d(rows1, CB, 128)

  def ones_init(i, _):
    ones[pl.ds(i * L, L)] = jnp.full((L,), 1.0, jnp.float32)
    return 0
  lax.fori_loop(0, CB // L, ones_init, 0)

  plsc.subcore_barrier()

  def keys(b):
    def g_(g, _):
      o = g * L
      kidx[b][pl.ds(o, L)] = ebd[b][pl.ds(o, L)] * R + ebr[b][pl.ds(o, L)]
      return 0
    lax.fori_loop(0, CB // L, g_, 0)

  def cnt_step(k, b):
    wait_dr(k, b)

    @pl.when(k >= 2)
    def _():
      pltpu.make_async_copy(ones, cnt_sh.at[kidx[b]], sem_c[b]).wait()
    keys(b)

    @pl.when(k + 2 < NCH)
    def _():
      issue_dr(k + 2, b)
    pltpu.async_copy(ones, cnt_sh.at[kidx[b]], sem_c[b], add=True)

  issue_dr(0, 0)
  issue_dr(1, 1)

  def cnt_pair(i, _):
    cnt_step(2 * i, 0)
    cnt_step(2 * i + 1, 1)
    return 0
  lax.fori_loop(0, NCH // 2, cnt_pair, 0)
  cnt_step(NCH - 1, 0)
  pltpu.make_async_copy(ones, cnt_sh.at[kidx0], sem_c0).wait()
  pltpu.make_async_copy(ones, cnt_sh.at[kidx1], sem_c1).wait()

  plsc.subcore_barrier()

  def front(k, b, wait_scat):
    if wait_scat:
      pltpu.make_async_copy(rows[b], agg_sh.at[didx[b]], sem_s[b]).wait()
    wait_edges(k, b)

    def idxg(g, _):
      o = g * L
      s16 = ebs[b][pl.ds(o, L)]
      r16 = ebr[b][pl.ds(o, L)]
      d16 = ebd[b][pl.ds(o, L)]
      kidx[b][pl.ds(o, L)] = d16 * R + r16
      gidx[b][pl.ds(o, L)] = (r16 * N + s16) * 2 + c
      didx[b][pl.ds(o, L)] = d16
      return 0
    lax.fori_loop(0, CB // L, idxg, 0)

    pltpu.async_copy(cnt_sh.at[kidx[b]], cv[b], sem_c[b])
    pltpu.async_copy(table.at[gidx[b]], rows[b], sem_r[b])

    @pl.when(k + 2 < NCH)
    def _():
      issue_edges(k + 2, b)

  def back(k, b):
    pltpu.make_async_copy(cnt_sh.at[kidx[b]], cv[b], sem_c[b]).wait()
    nb = k * CB

    def nrm_(g, _):
      o = g * L
      normall[pl.ds(nb + o, L)] = 1.0 / jnp.maximum(cv[b][pl.ds(o, L)], 1.0)
      return 0
    lax.fori_loop(0, CB // L, nrm_, 0)
    pltpu.make_async_copy(table.at[gidx[b]], rows[b], sem_r[b]).wait()
    _scale_rows(rows[b], normall, nb, 128)
    pltpu.async_copy(rows[b], agg_sh.at[didx[b]], sem_s[b], add=True)

  issue_edges(0, 0)
  issue_edges(1, 1)
  front(0, 0, wait_scat=False)
  front(1, 1, wait_scat=False)

  def main_pair(i, _):
    k = 2 * i
    back(k, 0)
    back(k + 1, 1)
    front(k + 2, 0, wait_scat=True)
    front(k + 3, 1, wait_scat=True)
    return 0
  lax.fori_loop(0, (NCH - 3) // 2, main_pair, 0)
  back(NCH - 3, 0)
  back(NCH - 2, 1)
  front(NCH - 1, 0, wait_scat=True)
  back(NCH - 1, 0)

  pltpu.make_async_copy(rows0, agg_sh.at[didx0], sem_s0).wait()
  pltpu.make_async_copy(rows1, agg_sh.at[didx1], sem_s1).wait()

  @pl.when(c == 0)
  def _():
    pltpu.sync_copy(normall, norm_out.at[pl.ds(s * ET, ET)])

  @pl.when(jnp.logical_and(c == 0, s == 0))
  def _():
    pltpu.sync_copy(zc.at[pl.ds(0, E2 - E)], norm_out.at[pl.ds(E, E2 - E)])

  plsc.subcore_barrier()
  _writeout(agg_sh, agg_out, rows0, c, s)


def _agg2_body(table, src_h, dst_h, rel_h, norm_h, agg_out,
               agg_sh,
               rows0, rows1, gidx0, gidx1, didx0, didx1,
               ebs0, ebd0, ebr0, ebs1, ebd1, ebr1, nrm0, nrm1,
               nrmv0, nrmv1,
               sem_e0, sem_e1, sem_r0, sem_r1, sem_s0, sem_s1):
  c = lax.axis_index("c")
  s = lax.axis_index("s")
  wid = c * NS + s
  rows = (rows0, rows1)
  gidx = (gidx0, gidx1)
  didx = (didx0, didx1)
  ebs = (ebs0, ebs1)
  ebd = (ebd0, ebd1)
  ebr = (ebr0, ebr1)
  nrm = (nrm0, nrm1)
  nrmv = (nrmv0, nrmv1)
  sem_e = (sem_e0, sem_e1)
  sem_r = (sem_r0, sem_r1)
  sem_s = (sem_s0, sem_s1)

  def eoff(k):
    return wid * ET2 + k * CB

  def eoffc(k):
    o = eoff(k)
    return jnp.where(o <= E - CB, o, 0)

  def issue_edges(k, b):
    o = eoffc(k)
    on = eoff(k)
    pltpu.async_copy(src_h.at[pl.ds(o, CB)], ebs[b], sem_e[b])
    pltpu.async_copy(dst_h.at[pl.ds(o, CB)], ebd[b], sem_e[b])
    pltpu.async_copy(rel_h.at[pl.ds(o, CB)], ebr[b], sem_e[b])
    pltpu.async_copy(norm_h.at[pl.ds(on, CB)], nrm[b], sem_e[b])

  def wait_edges(k, b):
    o = eoffc(k)
    on = eoff(k)
    pltpu.make_async_copy(src_h.at[pl.ds(o, CB)], ebs[b], sem_e[b]).wait()
    pltpu.make_async_copy(dst_h.at[pl.ds(o, CB)], ebd[b], sem_e[b]).wait()
    pltpu.make_async_copy(rel_h.at[pl.ds(o, CB)], ebr[b], sem_e[b]).wait()
    pltpu.make_async_copy(norm_h.at[pl.ds(on, CB)], nrm[b], sem_e[b]).wait()

  _zero_phase(agg_sh, rows0, 128, s)
  _zero_2d(rows1, CB, 128)
  plsc.subcore_barrier()

  def front(k, b, wait_scat):
    if wait_scat:
      pltpu.make_async_copy(rows[b], agg_sh.at[didx[b]], sem_s[b]).wait()
    wait_edges(k, b)

    def idxg(g, _):
      o = g * L
      s16 = ebs[b][pl.ds(o, L)]
      r16 = ebr[b][pl.ds(o, L)]
      gidx[b][pl.ds(o, L)] = r16 * N + s16
      didx[b][pl.ds(o, L)] = ebd[b][pl.ds(o, L)]
      nrmv[b][pl.ds(o, L)] = nrm[b][pl.ds(o, L)]
      return 0
    lax.fori_loop(0, CB // L, idxg, 0)

    pltpu.async_copy(table.at[gidx[b]], rows[b], sem_r[b])

    @pl.when(k + 2 < NCH2)
    def _():
      issue_edges(k + 2, b)

  def back(k, b):
    pltpu.make_async_copy(table.at[gidx[b]], rows[b], sem_r[b]).wait()
    _scale_rows(rows[b], nrmv[b], 0, OUT)
    pltpu.async_copy(rows[b], agg_sh.at[didx[b]], sem_s[b], add=True)

  issue_edges(0, 0)
  issue_edges(1, 1)
  front(0, 0, wait_scat=False)
  front(1, 1, wait_scat=False)

  def main_pair(i, _):
    k = 2 * i
    back(k, 0)
    back(k + 1, 1)
    front(k + 2, 0, wait_scat=True)
    front(k + 3, 1, wait_scat=True)
    return 0
  lax.fori_loop(0, (NCH2 - 2) // 2, main_pair, 0)
  back(NCH2 - 2, 0)
  back(NCH2 - 1, 1)

  pltpu.make_async_copy(rows0, agg_sh.at[didx0], sem_s0).wait()
  pltpu.make_async_copy(rows1, agg_sh.at[didx1], sem_s1).wait()

  plsc.subcore_barrier()
  _writeout(agg_sh, agg_out, rows0, c, s)


_MESH = plsc.VectorSubcoreMesh(
    core_axis_name="c", subcore_axis_name="s", num_cores=NC, num_subcores=NS)

_agg1 = pl.kernel(
    _agg1_body,
    out_type=(jax.ShapeDtypeStruct((NC, N, 128), jnp.float32),
              jax.ShapeDtypeStruct((E2,), jnp.float32)),
    mesh=_MESH,
    scratch_types=[
        pltpu.VMEM_SHARED((N, 128), jnp.float32),
        pltpu.VMEM_SHARED((N * R,), jnp.float32),
        pltpu.VMEM((CB, 128), jnp.float32),
        pltpu.VMEM((CB, 128), jnp.float32),
        pltpu.VMEM((CB,), jnp.int32),
        pltpu.VMEM((CB,), jnp.int32),
        pltpu.VMEM((CB,), jnp.int32),
        pltpu.VMEM((CB,), jnp.int32),
        pltpu.VMEM((CB,), jnp.int32),
        pltpu.VMEM((CB,), jnp.int32),
        pltpu.VMEM((CB,), jnp.int32),
        pltpu.VMEM((CB,), jnp.int32),
        pltpu.VMEM((CB,), jnp.int32),
        pltpu.VMEM((CB,), jnp.int32),
        pltpu.VMEM((CB,), jnp.int32),
        pltpu.VMEM((CB,), jnp.int32),
        pltpu.VMEM((CB,), jnp.float32),
        pltpu.VMEM((ET,), jnp.float32),
        pltpu.VMEM((CB,), jnp.float32),
        pltpu.VMEM((CB,), jnp.float32),
        pltpu.VMEM((5008,), jnp.float32),
        pltpu.SemaphoreType.DMA,
        pltpu.SemaphoreType.DMA,
        pltpu.SemaphoreType.DMA,
        pltpu.SemaphoreType.DMA,
        pltpu.SemaphoreType.DMA,
        pltpu.SemaphoreType.DMA,
        pltpu.SemaphoreType.DMA,
        pltpu.SemaphoreType.DMA,
    ],
)

_agg2 = pl.kernel(
    _agg2_body,
    out_type=jax.ShapeDtypeStruct((NC, N, 128), jnp.float32),
    mesh=_MESH,
    scratch_types=[
        pltpu.VMEM_SHARED((N, 128), jnp.float32),
        pltpu.VMEM((CB, 128), jnp.float32),
        pltpu.VMEM((CB, 128), jnp.float32),
        pltpu.VMEM((CB,), jnp.int32),
        pltpu.VMEM((CB,), jnp.int32),
        pltpu.VMEM((CB,), jnp.int32),
        pltpu.VMEM((CB,), jnp.int32),
        pltpu.VMEM((CB,), jnp.int32),
        pltpu.VMEM((CB,), jnp.int32),
        pltpu.VMEM((CB,), jnp.int32),
        pltpu.VMEM((CB,), jnp.int32),
        pltpu.VMEM((CB,), jnp.int32),
        pltpu.VMEM((CB,), jnp.int32),
        pltpu.VMEM((CB,), jnp.float32),
        pltpu.VMEM((CB,), jnp.float32),
        pltpu.VMEM((CB,), jnp.float32),
        pltpu.VMEM((CB,), jnp.float32),
        pltpu.SemaphoreType.DMA,
        pltpu.SemaphoreType.DMA,
        pltpu.SemaphoreType.DMA,
        pltpu.SemaphoreType.DMA,
        pltpu.SemaphoreType.DMA,
        pltpu.SemaphoreType.DMA,
    ],
)


def _mm_body(fpad, x_ref, w_ref, o_ref):
  acc = jnp.dot(x_ref[...].astype(jnp.bfloat16),
                w_ref[0].astype(jnp.bfloat16),
                preferred_element_type=jnp.float32)
  f = acc.shape[1]
  if fpad > f:
    o_ref[:, :f] = acc
    o_ref[:, f:] = jnp.zeros((acc.shape[0], fpad - f), jnp.float32)
  else:
    o_ref[...] = acc


def _mm(x, w, fpad):
  RW, Dd, F = w.shape
  Bn = 1000
  nb = N // Bn
  return pl.pallas_call(
      functools.partial(_mm_body, fpad),
      grid=(nb, RW),
      in_specs=[
          pl.BlockSpec((Bn, Dd), lambda i, r: (i, 0)),
          pl.BlockSpec((1, Dd, F), lambda i, r: (r, 0, 0)),
      ],
      out_specs=pl.BlockSpec((Bn, fpad), lambda i, r: (r * (N // 1000) + i, 0)),
      out_shape=jax.ShapeDtypeStruct((RW * N, fpad), jnp.float32),
  )(x, w)


def _ew_body(relu, fsum, a0_ref, a1_ref, x_ref, ws_ref, b_ref, o_ref):
  if fsum:
    agg = jnp.concatenate([a0_ref[0], a1_ref[0]], axis=1)
  else:
    agg = a0_ref[0][:, :OUT] + a1_ref[0][:, :OUT]
  v = agg + jnp.dot(x_ref[...].astype(jnp.bfloat16),
                    ws_ref[...].astype(jnp.bfloat16),
                    preferred_element_type=jnp.float32) + b_ref[...]
  if relu:
    v = jnp.maximum(v, 0.0)
  o_ref[...] = v


def _ew(agg, x, ws, b, f, relu, fsum):
  Bn = 1000
  nb = N // Bn
  return pl.pallas_call(
      functools.partial(_ew_body, relu, fsum),
      grid=(nb,),
      in_specs=[
          pl.BlockSpec((1, Bn, 128), lambda i: (0, i, 0)),
          pl.BlockSpec((1, Bn, 128), lambda i: (1, i, 0)),
          pl.BlockSpec((Bn, D), lambda i: (i, 0)),
          pl.BlockSpec((D, f), lambda i: (0, 0)),
          pl.BlockSpec((1, f), lambda i: (0, 0)),
      ],
      out_specs=pl.BlockSpec((Bn, f), lambda i: (i, 0)),
      out_shape=jax.ShapeDtypeStruct((N, f), jnp.float32),
  )(agg, agg, x, ws, b)


@jax.jit
def kernel(edge_index, edge_type, emb, W1, Wself1, b1, W2, Wself2, b2):
  src = edge_index[0]
  dst = edge_index[1]
  rel = edge_type

  xt1 = _mm(emb, W1, D)
  table1 = xt1.reshape(2 * R * N, D // 2)
  agg1, norm = _agg1(table1, src, dst, rel)
  h = _ew(agg1, emb, Wself1, b1.reshape(1, D), D,
          relu=True, fsum=True)

  xt2 = _mm(h, W2, 128)
  agg2 = _agg2(xt2, src, dst, rel, norm)
  out = _ew(agg2, h, Wself2, b2.reshape(1, OUT), OUT,
            relu=False, fsum=False)
  return out


# speedup vs baseline: 1.0084x; 1.0084x over previous
import functools

import jax
import jax.numpy as jnp
from jax import lax
from jax.experimental import pallas as pl
from jax.experimental.pallas import tpu as pltpu
from jax.experimental.pallas import tpu_sc as plsc

N = 10000
E = 160000
R = 8
D = 256
OUT = 64

NC = 2
NS = 16
L = 16

E2 = 163840
ET2 = E2 // (NC * NS)
ET = E // NS
CB = 80
CB2 = 128
NCH = ET // CB
NCH2 = ET2 // CB2
RB = 624
CPT = (N * R) // NS


def _splat(v, j):
  idx = jnp.full((L, 1), j, jnp.int32)
  dn = lax.GatherDimensionNumbers(
      offset_dims=(), collapsed_slice_dims=(0,), start_index_map=(0,))
  return lax.gather(v, idx, dn, (1,),
                    mode=lax.GatherScatterMode.PROMISE_IN_BOUNDS)


def _zero_2d(ref, nrows, ncols):
  z = jnp.zeros((L,), jnp.float32)

  def row(i, _):
    def col(k, _):
      ref[i, pl.ds(k * L, L)] = z
      return 0
    return lax.fori_loop(0, ncols // L, col, 0)
  lax.fori_loop(0, nrows, row, 0)


def _zero_1d(ref, n):
  z = jnp.zeros((L,), jnp.float32)

  def body(i, _):
    ref[pl.ds(i * L, L)] = z
    return 0
  lax.fori_loop(0, n // L, body, 0)


def _scale_rows(rows, nrmref, base, fsc, ng=CB // L):
  def group(g, _):
    n16 = nrmref[pl.ds(base + g * L, L)]
    for j in range(L):
      e = g * L + j
      ns = _splat(n16, j)
      for f in range(fsc // L):
        rows[e, pl.ds(f * L, L)] = rows[e, pl.ds(f * L, L)] * ns
    return 0
  lax.fori_loop(0, ng, group, 0)


def _row_range(s):
  base = s * RB
  n80 = jnp.where(s == NS - 1, 8, 7)
  tail = jnp.where(s == NS - 1, 0, 64)
  return base, n80, tail


def _zero_phase(agg_sh, rows, fh, s, cnt_sh=None, zc=None):
  _zero_2d(rows, CB, fh)
  base, n80, tail = _row_range(s)

  def body(k, _):
    pltpu.sync_copy(rows.at[pl.ds(0, CB)],
                    agg_sh.at[pl.ds(base + k * CB, CB)])
    return 0
  lax.fori_loop(0, n80, body, 0)

  @pl.when(tail > 0)
  def _():
    pltpu.sync_copy(rows.at[pl.ds(0, 64)],
                    agg_sh.at[pl.ds(base + 7 * CB, 64)])
  if cnt_sh is not None:
    _zero_1d(zc, 5008)
    pltpu.sync_copy(zc.at[pl.ds(0, CPT)], cnt_sh.at[pl.ds(s * CPT, CPT)])


def _writeout(agg_sh, agg_out, rows, c, s):
  base, n80, tail = _row_range(s)

  def body(k, _):
    o = base + k * CB
    pltpu.sync_copy(agg_sh.at[pl.ds(o, CB)], rows.at[pl.ds(0, CB)])
    pltpu.sync_copy(rows.at[pl.ds(0, CB)], agg_out.at[c, pl.ds(o, CB)])
    return 0
  lax.fori_loop(0, n80, body, 0)

  @pl.when(tail > 0)
  def _():
    o = base + 7 * CB
    pltpu.sync_copy(agg_sh.at[pl.ds(o, 64)], rows.at[pl.ds(0, 64)])
    pltpu.sync_copy(rows.at[pl.ds(0, 64)], agg_out.at[c, pl.ds(o, 64)])


def _agg1_body(table, src_h, dst_h, rel_h, agg_out, norm_out,
               agg_sh, cnt_sh,
               rows0, rows1, gidx0, gidx1, didx0, didx1, kidx0, kidx1,
               ebs0, ebd0, ebr0, ebs1, ebd1, ebr1,
               ones, normall, cv0, cv1, zc,
               sem_e0, sem_e1, sem_r0, sem_r1, sem_s0, sem_s1,
               sem_c0, sem_c1):
  c = lax.axis_index("c")
  s = lax.axis_index("s")
  rows = (rows0, rows1)
  gidx = (gidx0, gidx1)
  didx = (didx0, didx1)
  kidx = (kidx0, kidx1)
  ebs = (ebs0, ebs1)
  ebd = (ebd0, ebd1)
  ebr = (ebr0, ebr1)
  cv = (cv0, cv1)
  sem_e = (sem_e0, sem_e1)
  sem_r = (sem_r0, sem_r1)
  sem_s = (sem_s0, sem_s1)
  sem_c = (sem_c0, sem_c1)

  def eoff(k):
    return s * ET + k * CB

  def issue_dr(k, b):
    o = eoff(k)
    pltpu.async_copy(dst_h.at[pl.ds(o, CB)], ebd[b], sem_e[b])
    pltpu.async_copy(rel_h.at[pl.ds(o, CB)], ebr[b], sem_e[b])

  def wait_dr(k, b):
    o = eoff(k)
    pltpu.make_async_copy(dst_h.at[pl.ds(o, CB)], ebd[b], sem_e[b]).wait()
    pltpu.make_async_copy(rel_h.at[pl.ds(o, CB)], ebr[b], sem_e[b]).wait()

  def issue_edges(k, b):
    o = eoff(k)
    pltpu.async_copy(src_h.at[pl.ds(o, CB)], ebs[b], sem_e[b])
    pltpu.async_copy(dst_h.at[pl.ds(o, CB)], ebd[b], sem_e[b])
    pltpu.async_copy(rel_h.at[pl.ds(o, CB)], ebr[b], sem_e[b])

  def wait_edges(k, b):
    o = eoff(k)
    pltpu.make_async_copy(src_h.at[pl.ds(o, CB)], ebs[b], sem_e[b]).wait()
    pltpu.make_async_copy(dst_h.at[pl.ds(o, CB)], ebd[b], sem_e[b]).wait()
    pltpu.make_async_copy(rel_h.at[pl.ds(o, CB)], ebr[b], sem_e[b]).wait()

  _zero_phase(agg_sh, rows0, 128, s, cnt_sh, zc)
  _zero_2d(rows1, CB, 128)

  def ones_init(i, _):
    ones[pl.ds(i * L, L)] = jnp.full((L,), 1.0, jnp.float32)
    return 0
  lax.fori_loop(0, CB // L, ones_init, 0)

  plsc.subcore_barrier()

  def keys(b):
    def g_(g, _):
      o = g * L
      kidx[b][pl.ds(o, L)] = ebd[b][pl.ds(o, L)] * R + ebr[b][pl.ds(o, L)]
      return 0
    lax.fori_loop(0, CB // L, g_, 0)

  def cnt_step(k, b):
    wait_dr(k, b)

    @pl.when(k >= 2)
    def _():
      pltpu.make_async_copy(ones, cnt_sh.at[kidx[b]], sem_c[b]).wait()
    keys(b)

    @pl.when(k + 2 < NCH)
    def _():
      issue_dr(k + 2, b)
    pltpu.async_copy(ones, cnt_sh.at[kidx[b]], sem_c[b], add=True)

  issue_dr(0, 0)
  issue_dr(1, 1)

  def cnt_pair(i, _):
    cnt_step(2 * i, 0)
    cnt_step(2 * i + 1, 1)
    return 0
  lax.fori_loop(0, NCH // 2, cnt_pair, 0)
  cnt_step(NCH - 1, 0)
  pltpu.make_async_copy(ones, cnt_sh.at[kidx0], sem_c0).wait()
  pltpu.make_async_copy(ones, cnt_sh.at[kidx1], sem_c1).wait()

  plsc.subcore_barrier()

  def front(k, b, wait_scat):
    if wait_scat:
      pltpu.make_async_copy(rows[b], agg_sh.at[didx[b]], sem_s[b]).wait()
    wait_edges(k, b)

    def idxg(g, _):
      o = g * L
      s16 = ebs[b][pl.ds(o, L)]
      r16 = ebr[b][pl.ds(o, L)]
      d16 = ebd[b][pl.ds(o, L)]
      kidx[b][pl.ds(o, L)] = d16 * R + r16
      gidx[b][pl.ds(o, L)] = (r16 * N + s16) * 2 + c
      didx[b][pl.ds(o, L)] = d16
      return 0
    lax.fori_loop(0, CB // L, idxg, 0)

    pltpu.async_copy(cnt_sh.at[kidx[b]], cv[b], sem_c[b])
    pltpu.async_copy(table.at[gidx[b]], rows[b], sem_r[b])

    @pl.when(k + 2 < NCH)
    def _():
      issue_edges(k + 2, b)

  def back(k, b):
    pltpu.make_async_copy(cnt_sh.at[kidx[b]], cv[b], sem_c[b]).wait()
    nb = k * CB

    def nrm_(g, _):
      o = g * L
      normall[pl.ds(nb + o, L)] = 1.0 / jnp.maximum(cv[b][pl.ds(o, L)], 1.0)
      return 0
    lax.fori_loop(0, CB // L, nrm_, 0)
    pltpu.make_async_copy(table.at[gidx[b]], rows[b], sem_r[b]).wait()
    _scale_rows(rows[b], normall, nb, 128)
    pltpu.async_copy(rows[b], agg_sh.at[didx[b]], sem_s[b], add=True)

  issue_edges(0, 0)
  issue_edges(1, 1)
  front(0, 0, wait_scat=False)
  front(1, 1, wait_scat=False)

  def main_pair(i, _):
    k = 2 * i
    back(k, 0)
    back(k + 1, 1)
    front(k + 2, 0, wait_scat=True)
    front(k + 3, 1, wait_scat=True)
    return 0
  lax.fori_loop(0, (NCH - 3) // 2, main_pair, 0)
  back(NCH - 3, 0)
  back(NCH - 2, 1)
  front(NCH - 1, 0, wait_scat=True)
  back(NCH - 1, 0)

  pltpu.make_async_copy(rows0, agg_sh.at[didx0], sem_s0).wait()
  pltpu.make_async_copy(rows1, agg_sh.at[didx1], sem_s1).wait()

  @pl.when(c == 0)
  def _():
    pltpu.sync_copy(normall, norm_out.at[pl.ds(s * ET, ET)])

  @pl.when(jnp.logical_and(c == 0, s == 0))
  def _():
    pltpu.sync_copy(zc.at[pl.ds(0, E2 - E)], norm_out.at[pl.ds(E, E2 - E)])

  plsc.subcore_barrier()
  _writeout(agg_sh, agg_out, rows0, c, s)


def _agg2_body(table, src_h, dst_h, rel_h, norm_h, agg_out,
               agg_sh,
               rows0, rows1, gidx0, gidx1, didx0, didx1,
               ebs0, ebd0, ebr0, ebs1, ebd1, ebr1, nrm0, nrm1,
               nrmv0, nrmv1,
               sem_e0, sem_e1, sem_r0, sem_r1, sem_s0, sem_s1):
  c = lax.axis_index("c")
  s = lax.axis_index("s")
  wid = c * NS + s
  rows = (rows0, rows1)
  gidx = (gidx0, gidx1)
  didx = (didx0, didx1)
  ebs = (ebs0, ebs1)
  ebd = (ebd0, ebd1)
  ebr = (ebr0, ebr1)
  nrm = (nrm0, nrm1)
  nrmv = (nrmv0, nrmv1)
  sem_e = (sem_e0, sem_e1)
  sem_r = (sem_r0, sem_r1)
  sem_s = (sem_s0, sem_s1)

  def eoff(k):
    return wid * ET2 + k * CB2

  def eoffc(k):
    o = eoff(k)
    return jnp.where(o <= E - CB2, o, 0)

  def issue_edges(k, b):
    o = eoffc(k)
    on = eoff(k)
    pltpu.async_copy(src_h.at[pl.ds(o, CB2)], ebs[b], sem_e[b])
    pltpu.async_copy(dst_h.at[pl.ds(o, CB2)], ebd[b], sem_e[b])
    pltpu.async_copy(rel_h.at[pl.ds(o, CB2)], ebr[b], sem_e[b])
    pltpu.async_copy(norm_h.at[pl.ds(on, CB2)], nrm[b], sem_e[b])

  def wait_edges(k, b):
    o = eoffc(k)
    on = eoff(k)
    pltpu.make_async_copy(src_h.at[pl.ds(o, CB2)], ebs[b], sem_e[b]).wait()
    pltpu.make_async_copy(dst_h.at[pl.ds(o, CB2)], ebd[b], sem_e[b]).wait()
    pltpu.make_async_copy(rel_h.at[pl.ds(o, CB2)], ebr[b], sem_e[b]).wait()
    pltpu.make_async_copy(norm_h.at[pl.ds(on, CB2)], nrm[b], sem_e[b]).wait()

  _zero_phase(agg_sh, rows0, 128, s)
  _zero_2d(rows1, CB, 128)
  plsc.subcore_barrier()

  def front(k, b, wait_scat):
    if wait_scat:
      pltpu.make_async_copy(rows[b], agg_sh.at[didx[b]], sem_s[b]).wait()
    wait_edges(k, b)

    def idxg(g, _):
      o = g * L
      s16 = ebs[b][pl.ds(o, L)]
      r16 = ebr[b][pl.ds(o, L)]
      gidx[b][pl.ds(o, L)] = r16 * N + s16
      didx[b][pl.ds(o, L)] = ebd[b][pl.ds(o, L)]
      nrmv[b][pl.ds(o, L)] = nrm[b][pl.ds(o, L)]
      return 0
    lax.fori_loop(0, CB2 // L, idxg, 0)

    pltpu.async_copy(table.at[gidx[b]], rows[b], sem_r[b])

    @pl.when(k + 2 < NCH2)
    def _():
      issue_edges(k + 2, b)

  def back(k, b):
    pltpu.make_async_copy(table.at[gidx[b]], rows[b], sem_r[b]).wait()
    _scale_rows(rows[b], nrmv[b], 0, OUT, ng=CB2 // L)
    pltpu.async_copy(rows[b], agg_sh.at[didx[b]], sem_s[b], add=True)

  issue_edges(0, 0)
  issue_edges(1, 1)
  front(0, 0, wait_scat=False)
  front(1, 1, wait_scat=False)

  def main_pair(i, _):
    k = 2 * i
    back(k, 0)
    back(k + 1, 1)
    front(k + 2, 0, wait_scat=True)
    front(k + 3, 1, wait_scat=True)
    return 0
  lax.fori_loop(0, (NCH2 - 2) // 2, main_pair, 0)
  back(NCH2 - 2, 0)
  back(NCH2 - 1, 1)

  pltpu.make_async_copy(rows0, agg_sh.at[didx0], sem_s0).wait()
  pltpu.make_async_copy(rows1, agg_sh.at[didx1], sem_s1).wait()

  plsc.subcore_barrier()
  _writeout(agg_sh, agg_out, rows0, c, s)


_MESH = plsc.VectorSubcoreMesh(
    core_axis_name="c", subcore_axis_name="s", num_cores=NC, num_subcores=NS)

_agg1 = pl.kernel(
    _agg1_body,
    out_type=(jax.ShapeDtypeStruct((NC, N, 128), jnp.float32),
              jax.ShapeDtypeStruct((E2,), jnp.float32)),
    mesh=_MESH,
    scratch_types=[
        pltpu.VMEM_SHARED((N, 128), jnp.float32),
        pltpu.VMEM_SHARED((N * R,), jnp.float32),
        pltpu.VMEM((CB, 128), jnp.float32),
        pltpu.VMEM((CB, 128), jnp.float32),
        pltpu.VMEM((CB,), jnp.int32),
        pltpu.VMEM((CB,), jnp.int32),
        pltpu.VMEM((CB,), jnp.int32),
        pltpu.VMEM((CB,), jnp.int32),
        pltpu.VMEM((CB,), jnp.int32),
        pltpu.VMEM((CB,), jnp.int32),
        pltpu.VMEM((CB,), jnp.int32),
        pltpu.VMEM((CB,), jnp.int32),
        pltpu.VMEM((CB,), jnp.int32),
        pltpu.VMEM((CB,), jnp.int32),
        pltpu.VMEM((CB,), jnp.int32),
        pltpu.VMEM((CB,), jnp.int32),
        pltpu.VMEM((CB,), jnp.float32),
        pltpu.VMEM((ET,), jnp.float32),
        pltpu.VMEM((CB,), jnp.float32),
        pltpu.VMEM((CB,), jnp.float32),
        pltpu.VMEM((5008,), jnp.float32),
        pltpu.SemaphoreType.DMA,
        pltpu.SemaphoreType.DMA,
        pltpu.SemaphoreType.DMA,
        pltpu.SemaphoreType.DMA,
        pltpu.SemaphoreType.DMA,
        pltpu.SemaphoreType.DMA,
        pltpu.SemaphoreType.DMA,
        pltpu.SemaphoreType.DMA,
    ],
)

_agg2 = pl.kernel(
    _agg2_body,
    out_type=jax.ShapeDtypeStruct((NC, N, 128), jnp.float32),
    mesh=_MESH,
    scratch_types=[
        pltpu.VMEM_SHARED((N, 128), jnp.float32),
        pltpu.VMEM((CB2, 128), jnp.float32),
        pltpu.VMEM((CB2, 128), jnp.float32),
        pltpu.VMEM((CB2,), jnp.int32),
        pltpu.VMEM((CB2,), jnp.int32),
        pltpu.VMEM((CB2,), jnp.int32),
        pltpu.VMEM((CB2,), jnp.int32),
        pltpu.VMEM((CB2,), jnp.int32),
        pltpu.VMEM((CB2,), jnp.int32),
        pltpu.VMEM((CB2,), jnp.int32),
        pltpu.VMEM((CB2,), jnp.int32),
        pltpu.VMEM((CB2,), jnp.int32),
        pltpu.VMEM((CB2,), jnp.int32),
        pltpu.VMEM((CB2,), jnp.float32),
        pltpu.VMEM((CB2,), jnp.float32),
        pltpu.VMEM((CB2,), jnp.float32),
        pltpu.VMEM((CB2,), jnp.float32),
        pltpu.SemaphoreType.DMA,
        pltpu.SemaphoreType.DMA,
        pltpu.SemaphoreType.DMA,
        pltpu.SemaphoreType.DMA,
        pltpu.SemaphoreType.DMA,
        pltpu.SemaphoreType.DMA,
    ],
)


def _mm_body(fpad, x_ref, w_ref, o_ref):
  acc = jnp.dot(x_ref[...].astype(jnp.bfloat16),
                w_ref[0].astype(jnp.bfloat16),
                preferred_element_type=jnp.float32)
  f = acc.shape[1]
  if fpad > f:
    o_ref[:, :f] = acc
    o_ref[:, f:] = jnp.zeros((acc.shape[0], fpad - f), jnp.float32)
  else:
    o_ref[...] = acc


def _mm(x, w, fpad):
  RW, Dd, F = w.shape
  Bn = 1000
  nb = N // Bn
  return pl.pallas_call(
      functools.partial(_mm_body, fpad),
      grid=(nb, RW),
      in_specs=[
          pl.BlockSpec((Bn, Dd), lambda i, r: (i, 0)),
          pl.BlockSpec((1, Dd, F), lambda i, r: (r, 0, 0)),
      ],
      out_specs=pl.BlockSpec((Bn, fpad), lambda i, r: (r * (N // 1000) + i, 0)),
      out_shape=jax.ShapeDtypeStruct((RW * N, fpad), jnp.float32),
  )(x, w)


def _ew_body(relu, fsum, a0_ref, a1_ref, x_ref, ws_ref, b_ref, o_ref):
  if fsum:
    agg = jnp.concatenate([a0_ref[0], a1_ref[0]], axis=1)
  else:
    agg = a0_ref[0][:, :OUT] + a1_ref[0][:, :OUT]
  v = agg + jnp.dot(x_ref[...].astype(jnp.bfloat16),
                    ws_ref[...].astype(jnp.bfloat16),
                    preferred_element_type=jnp.float32) + b_ref[...]
  if relu:
    v = jnp.maximum(v, 0.0)
  o_ref[...] = v


def _ew(agg, x, ws, b, f, relu, fsum):
  Bn = 1000
  nb = N // Bn
  return pl.pallas_call(
      functools.partial(_ew_body, relu, fsum),
      grid=(nb,),
      in_specs=[
          pl.BlockSpec((1, Bn, 128), lambda i: (0, i, 0)),
          pl.BlockSpec((1, Bn, 128), lambda i: (1, i, 0)),
          pl.BlockSpec((Bn, D), lambda i: (i, 0)),
          pl.BlockSpec((D, f), lambda i: (0, 0)),
          pl.BlockSpec((1, f), lambda i: (0, 0)),
      ],
      out_specs=pl.BlockSpec((Bn, f), lambda i: (i, 0)),
      out_shape=jax.ShapeDtypeStruct((N, f), jnp.float32),
  )(agg, agg, x, ws, b)


@jax.jit
def kernel(edge_index, edge_type, emb, W1, Wself1, b1, W2, Wself2, b2):
  src = edge_index[0]
  dst = edge_index[1]
  rel = edge_type

  xt1 = _mm(emb, W1, D)
  table1 = xt1.reshape(2 * R * N, D // 2)
  agg1, norm = _agg1(table1, src, dst, rel)
  h = _ew(agg1, emb, Wself1, b1.reshape(1, D), D,
          relu=True, fsum=True)

  xt2 = _mm(h, W2, 128)
  agg2 = _agg2(xt2, src, dst, rel, norm)
  out = _ew(agg2, h, Wself2, b2.reshape(1, OUT), OUT,
            relu=False, fsum=False)
  return out
